# baseline (speedup 1.0000x reference)
import functools

import jax
import jax.numpy as jnp
import numpy as np
from jax import lax
from jax.experimental import pallas as pl
from jax.experimental.pallas import tpu as pltpu
from jax.experimental.pallas import tpu_sc as plsc

_B = 1024
_P = 64
_ROWS = _B * _P
_NW = 32
_RPW = _ROWS // _NW
_L = 16
_STEPS = _RPW // _L

_SPACE = np.array([8000.0, 8000.0, 2000.0], np.float32)
_VOX = np.array([80.0, 80.0, 20.0], np.float32)
_CENTER = np.array([0.0, 0.0, 1000.0], np.float32)
_SCALE = _SPACE / (_VOX - 1.0)
_BIAS = _CENTER - _SPACE / 2.0
_MIN_SCORE = 0.3


@functools.partial(
    pl.kernel,
    mesh=plsc.VectorSubcoreMesh(core_axis_name="c", subcore_axis_name="s"),
    out_type=jax.ShapeDtypeStruct((_ROWS * 7,), jnp.float32),
    scratch_types=[
        pltpu.VMEM((_RPW * 3,), jnp.int32),
        pltpu.VMEM((_RPW,), jnp.float32),
        pltpu.VMEM((_RPW * 2,), jnp.float32),
        pltpu.VMEM((_RPW * 7,), jnp.float32),
    ],
    compiler_params=pltpu.CompilerParams(needs_layout_passes=False),
)
def _proposal_sc(idx_hbm, conf_hbm, bbox_hbm, out_hbm, idx_v, conf_v, bbox_v, out_v):
    wid = lax.axis_index("s") * 2 + lax.axis_index("c")
    row0 = wid * _RPW
    pltpu.sync_copy(idx_hbm.at[pl.ds(row0 * 3, _RPW * 3)], idx_v)
    pltpu.sync_copy(conf_hbm.at[pl.ds(row0, _RPW)], conf_v)
    pltpu.sync_copy(bbox_hbm.at[pl.ds(row0 * 2, _RPW * 2)], bbox_v)

    sx, sy, sz = float(_SCALE[0]), float(_SCALE[1]), float(_SCALE[2])
    bx, by, bz = float(_BIAS[0]), float(_BIAS[1]), float(_BIAS[2])

    def step(i, carry):
        r0 = i * _L
        r = r0 + lax.iota(jnp.int32, _L)
        i3 = r * 3
        ix = plsc.load_gather(idx_v, [i3])
        iy = plsc.load_gather(idx_v, [i3 + 1])
        iz = plsc.load_gather(idx_v, [i3 + 2])
        cf = conf_v[pl.ds(r0, _L)]
        i2 = r * 2
        b0 = plsc.load_gather(bbox_v, [i2])
        b1 = plsc.load_gather(bbox_v, [i2 + 1])
        x = ix.astype(jnp.float32) * sx + bx
        y = iy.astype(jnp.float32) * sy + by
        z = iz.astype(jnp.float32) * sz + bz
        fl = (cf > _MIN_SCORE).astype(jnp.float32) - 1.0
        o = r * 7
        plsc.store_scatter(out_v, [o], x)
        plsc.store_scatter(out_v, [o + 1], y)
        plsc.store_scatter(out_v, [o + 2], z)
        plsc.store_scatter(out_v, [o + 3], fl)
        plsc.store_scatter(out_v, [o + 4], cf)
        plsc.store_scatter(out_v, [o + 5], b0)
        plsc.store_scatter(out_v, [o + 6], b1)
        return carry

    lax.fori_loop(0, _STEPS, step, 0)
    pltpu.sync_copy(out_v, out_hbm.at[pl.ds(row0 * 7, _RPW * 7)])


def kernel(topk_index, topk_confs, match_bbox_preds, meta):
    del meta
    idx = topk_index.reshape(-1)
    conf = topk_confs.reshape(-1)
    bbox = match_bbox_preds.reshape(-1)
    out = _proposal_sc(idx, conf, bbox)
    return out.reshape(_B, _P, 7)


# speedup vs baseline: 1.0135x; 1.0135x over previous
import functools

import jax
import jax.numpy as jnp
import numpy as np
from jax import lax
from jax.experimental import pallas as pl
from jax.experimental.pallas import tpu as pltpu
from jax.experimental.pallas import tpu_sc as plsc

_B = 1024
_P = 64
_ROWS = _B * _P
_NW = 32
_RPW = _ROWS // _NW
_L = 16
_STEPS = _RPW // _L

_SPACE = np.array([8000.0, 8000.0, 2000.0], np.float32)
_VOX = np.array([80.0, 80.0, 20.0], np.float32)
_CENTER = np.array([0.0, 0.0, 1000.0], np.float32)
_SCALE = _SPACE / (_VOX - 1.0)
_BIAS = _CENTER - _SPACE / 2.0
_MIN_SCORE = 0.3


@functools.partial(
    pl.kernel,
    mesh=plsc.VectorSubcoreMesh(core_axis_name="c", subcore_axis_name="s"),
    out_type=jax.ShapeDtypeStruct((_ROWS * 7,), jnp.float32),
    scratch_types=[
        pltpu.VMEM((_RPW * 3,), jnp.int32),
        pltpu.VMEM((_RPW,), jnp.float32),
        pltpu.VMEM((_RPW * 2,), jnp.float32),
        pltpu.VMEM((_RPW * 7,), jnp.float32),
    ],
    compiler_params=pltpu.CompilerParams(
        needs_layout_passes=False,
        skip_device_barrier=True,
        disable_bounds_checks=True,
        disable_semaphore_checks=True,
    ),
)
def _proposal_sc(idx_hbm, conf_hbm, bbox_hbm, out_hbm, idx_v, conf_v, bbox_v, out_v):
    wid = lax.axis_index("s") * 2 + lax.axis_index("c")
    row0 = wid * _RPW
    pltpu.sync_copy(idx_hbm.at[pl.ds(row0 * 3, _RPW * 3)], idx_v)
    pltpu.sync_copy(conf_hbm.at[pl.ds(row0, _RPW)], conf_v)
    pltpu.sync_copy(bbox_hbm.at[pl.ds(row0 * 2, _RPW * 2)], bbox_v)

    sx, sy, sz = float(_SCALE[0]), float(_SCALE[1]), float(_SCALE[2])
    bx, by, bz = float(_BIAS[0]), float(_BIAS[1]), float(_BIAS[2])

    def step(i, carry):
        r0 = i * _L
        r = r0 + lax.iota(jnp.int32, _L)
        i3 = r * 3
        ix = plsc.load_gather(idx_v, [i3])
        iy = plsc.load_gather(idx_v, [i3 + 1])
        iz = plsc.load_gather(idx_v, [i3 + 2])
        cf = conf_v[pl.ds(r0, _L)]
        i2 = r * 2
        b0 = plsc.load_gather(bbox_v, [i2])
        b1 = plsc.load_gather(bbox_v, [i2 + 1])
        x = ix.astype(jnp.float32) * sx + bx
        y = iy.astype(jnp.float32) * sy + by
        z = iz.astype(jnp.float32) * sz + bz
        fl = (cf > _MIN_SCORE).astype(jnp.float32) - 1.0
        o = r * 7
        plsc.store_scatter(out_v, [o], x)
        plsc.store_scatter(out_v, [o + 1], y)
        plsc.store_scatter(out_v, [o + 2], z)
        plsc.store_scatter(out_v, [o + 3], fl)
        plsc.store_scatter(out_v, [o + 4], cf)
        plsc.store_scatter(out_v, [o + 5], b0)
        plsc.store_scatter(out_v, [o + 6], b1)
        return carry

    lax.fori_loop(0, _STEPS, step, 0)
    pltpu.sync_copy(out_v, out_hbm.at[pl.ds(row0 * 7, _RPW * 7)])


def kernel(topk_index, topk_confs, match_bbox_preds, meta):
    del meta
    idx = topk_index.reshape(-1)
    conf = topk_confs.reshape(-1)
    bbox = match_bbox_preds.reshape(-1)
    out = _proposal_sc(idx, conf, bbox)
    return out.reshape(_B, _P, 7)


# speedup vs baseline: 31.7944x; 31.3724x over previous
import functools

import jax
import jax.numpy as jnp
import numpy as np
from jax.experimental import pallas as pl
from jax.experimental.pallas import tpu as pltpu

_B = 1024
_P = 64

_SPACE = np.array([8000.0, 8000.0, 2000.0], np.float32)
_VOX = np.array([80.0, 80.0, 20.0], np.float32)
_CENTER = np.array([0.0, 0.0, 1000.0], np.float32)
_SCALE = _SPACE / (_VOX - 1.0)
_BIAS = _CENTER - _SPACE / 2.0
_MIN_SCORE = 0.3

_PB = 8
_GRID = _P // _PB


def _body(idx_ref, conf_ref, bbox_ref, out_ref):
    sx, sy, sz = float(_SCALE[0]), float(_SCALE[1]), float(_SCALE[2])
    bx, by, bz = float(_BIAS[0]), float(_BIAS[1]), float(_BIAS[2])
    idxf = idx_ref[...].astype(jnp.float32)
    out_ref[0] = idxf[0] * sx + bx
    out_ref[1] = idxf[1] * sy + by
    out_ref[2] = idxf[2] * sz + bz
    cf = conf_ref[...]
    out_ref[3] = (cf > _MIN_SCORE).astype(jnp.float32) - 1.0
    out_ref[4] = cf
    out_ref[5] = bbox_ref[:, 0, :]
    out_ref[6] = bbox_ref[:, 1, :]


@jax.jit
def _proposal_tc(idx_t, conf_t, bbox_t):
    return pl.pallas_call(
        _body,
        grid=(_GRID,),
        in_specs=[
            pl.BlockSpec((3, _PB, _B), lambda i: (0, i, 0)),
            pl.BlockSpec((_PB, _B), lambda i: (i, 0)),
            pl.BlockSpec((_PB, 2, _B), lambda i: (i, 0, 0)),
        ],
        out_specs=pl.BlockSpec((7, _PB, _B), lambda i: (0, i, 0)),
        out_shape=jax.ShapeDtypeStruct((7, _P, _B), jnp.float32),
    )(idx_t, conf_t, bbox_t)


def kernel(topk_index, topk_confs, match_bbox_preds, meta):
    del meta
    idx_t = jnp.transpose(topk_index, (2, 1, 0))
    conf_t = jnp.transpose(topk_confs, (1, 0))
    bbox_t = jnp.transpose(match_bbox_preds, (1, 2, 0))
    out_t = _proposal_tc(idx_t, conf_t, bbox_t)
    return jnp.transpose(out_t, (2, 1, 0))


# speedup vs baseline: 47.6745x; 1.4995x over previous
import functools

import jax
import jax.numpy as jnp
import numpy as np
from jax.experimental import pallas as pl
from jax.experimental.pallas import tpu as pltpu

_B = 1024
_P = 64

_SPACE = np.array([8000.0, 8000.0, 2000.0], np.float32)
_VOX = np.array([80.0, 80.0, 20.0], np.float32)
_CENTER = np.array([0.0, 0.0, 1000.0], np.float32)
_SCALE = _SPACE / (_VOX - 1.0)
_BIAS = _CENTER - _SPACE / 2.0
_MIN_SCORE = 0.3

_PB = 16
_GRID = _P // _PB


def _body(idx_ref, conf_ref, bbox_ref, out_ref):
    sx, sy, sz = float(_SCALE[0]), float(_SCALE[1]), float(_SCALE[2])
    bx, by, bz = float(_BIAS[0]), float(_BIAS[1]), float(_BIAS[2])
    idxf = idx_ref[...].astype(jnp.float32)
    out_ref[0] = idxf[0] * sx + bx
    out_ref[1] = idxf[1] * sy + by
    out_ref[2] = idxf[2] * sz + bz
    cf = conf_ref[...]
    out_ref[3] = (cf > _MIN_SCORE).astype(jnp.float32) - 1.0
    out_ref[4] = cf
    out_ref[5] = bbox_ref[:, 0, :]
    out_ref[6] = bbox_ref[:, 1, :]


@jax.jit
def _proposal_tc(idx_t, conf_t, bbox_t):
    return pl.pallas_call(
        _body,
        grid=(_GRID,),
        in_specs=[
            pl.BlockSpec((3, _PB, _B), lambda i: (0, i, 0)),
            pl.BlockSpec((_PB, _B), lambda i: (i, 0)),
            pl.BlockSpec((_PB, 2, _B), lambda i: (i, 0, 0)),
        ],
        out_specs=pl.BlockSpec((7, _PB, _B), lambda i: (0, i, 0)),
        out_shape=jax.ShapeDtypeStruct((7, _P, _B), jnp.float32),
    )(idx_t, conf_t, bbox_t)


def kernel(topk_index, topk_confs, match_bbox_preds, meta):
    del meta
    idx_t = jnp.transpose(topk_index, (2, 1, 0))
    conf_t = jnp.transpose(topk_confs, (1, 0))
    bbox_t = jnp.transpose(match_bbox_preds, (1, 2, 0))
    out_t = _proposal_tc(idx_t, conf_t, bbox_t)
    return jnp.transpose(out_t, (2, 1, 0))


# speedup vs baseline: 66.9970x; 1.4053x over previous
import functools

import jax
import jax.numpy as jnp
import numpy as np
from jax.experimental import pallas as pl
from jax.experimental.pallas import tpu as pltpu

_B = 1024
_P = 64

_SPACE = np.array([8000.0, 8000.0, 2000.0], np.float32)
_VOX = np.array([80.0, 80.0, 20.0], np.float32)
_CENTER = np.array([0.0, 0.0, 1000.0], np.float32)
_SCALE = _SPACE / (_VOX - 1.0)
_BIAS = _CENTER - _SPACE / 2.0
_MIN_SCORE = 0.3

_PB = 32
_GRID = _P // _PB


def _body(idx_ref, conf_ref, bbox_ref, out_ref):
    sx, sy, sz = float(_SCALE[0]), float(_SCALE[1]), float(_SCALE[2])
    bx, by, bz = float(_BIAS[0]), float(_BIAS[1]), float(_BIAS[2])
    idxf = idx_ref[...].astype(jnp.float32)
    out_ref[0] = idxf[0] * sx + bx
    out_ref[1] = idxf[1] * sy + by
    out_ref[2] = idxf[2] * sz + bz
    cf = conf_ref[...]
    out_ref[3] = (cf > _MIN_SCORE).astype(jnp.float32) - 1.0
    out_ref[4] = cf
    out_ref[5] = bbox_ref[:, 0, :]
    out_ref[6] = bbox_ref[:, 1, :]


@jax.jit
def _proposal_tc(idx_t, conf_t, bbox_t):
    return pl.pallas_call(
        _body,
        grid=(_GRID,),
        in_specs=[
            pl.BlockSpec((3, _PB, _B), lambda i: (0, i, 0)),
            pl.BlockSpec((_PB, _B), lambda i: (i, 0)),
            pl.BlockSpec((_PB, 2, _B), lambda i: (i, 0, 0)),
        ],
        out_specs=pl.BlockSpec((7, _PB, _B), lambda i: (0, i, 0)),
        out_shape=jax.ShapeDtypeStruct((7, _P, _B), jnp.float32),
    )(idx_t, conf_t, bbox_t)


def kernel(topk_index, topk_confs, match_bbox_preds, meta):
    del meta
    idx_t = jnp.transpose(topk_index, (2, 1, 0))
    conf_t = jnp.transpose(topk_confs, (1, 0))
    bbox_t = jnp.transpose(match_bbox_preds, (1, 2, 0))
    out_t = _proposal_tc(idx_t, conf_t, bbox_t)
    return jnp.transpose(out_t, (2, 1, 0))
